# Initial kernel scaffold; baseline (speedup 1.0000x reference)
import jax
import jax.numpy as jnp
from jax.experimental import pallas as pl


def kernel(x, edge_index, W1_self, W1_neigh, b1, W2_self, W2_neigh, b2):
    raise NotImplementedError("write your pallas kernel here")


# speedup vs baseline: 12.3700x; 12.3700x over previous
import functools

import jax
import jax.numpy as jnp
from jax import lax
from jax.experimental import pallas as pl
from jax.experimental.pallas import tpu as pltpu, tpu_sc as plsc

N = 10000
E = 320000
D = 128

NC = 2
NS = 16
NW = NC * NS
ROWS_PER_TILE = 640
N_PAD = NS * ROWS_PER_TILE
CHUNK = 128
CHUNKS = 80
E_TILE = CHUNKS * CHUNK
E_PAD = NW * E_TILE
_SHIFT = 14
_MASK = (1 << _SHIFT) - 1

_f32 = jnp.float32
_i32 = jnp.int32


def _mesh():
    return plsc.VectorSubcoreMesh(core_axis_name="c", subcore_axis_name="s",
                                  num_cores=NC, num_subcores=NS)


def _unpack_dst(pk, j, dbuf):
    for c in range(CHUNK // 16):
        sl = pl.ds(c * 16, 16)
        dbuf[0, sl] = lax.shift_right_logical(pk[j, sl], _SHIFT)


def _unpack_both(pk, j, sbuf, dbuf):
    for c in range(CHUNK // 16):
        sl = pl.ds(c * 16, 16)
        v = pk[j, sl]
        sbuf[0, sl] = v & _MASK
        dbuf[0, sl] = lax.shift_right_logical(v, _SHIFT)


def _deg_body(pkI, invd_out, pk_a, pk_b, d128, ones_v, degv, invrep, deg_sp):
    cid = lax.axis_index("c")
    sid = lax.axis_index("s")
    row0 = sid * ROWS_PER_TILE

    zero16 = jnp.zeros((16,), _f32)
    one16 = jnp.ones((16,), _f32)

    def _zd(i, _):
        degv[pl.ds(i * 16, 16)] = zero16
        return 0
    lax.fori_loop(0, ROWS_PER_TILE // 16, _zd, 0)
    pltpu.sync_copy(degv, deg_sp.at[pl.ds(row0, ROWS_PER_TILE)])

    def _ob(i, _):
        ones_v[pl.ds(i * 16, 16)] = one16
        return 0
    lax.fori_loop(0, CHUNK // 16, _ob, 0)

    pltpu.sync_copy(pkI.at[sid * NC + cid], pk_a)
    pltpu.sync_copy(pkI.at[sid * NC + (1 - cid)], pk_b)

    plsc.subcore_barrier()

    def _step(j, _):
        _unpack_dst(pk_a, j, d128)
        pltpu.sync_copy(ones_v, deg_sp.at[d128.at[0]], add=True)
        _unpack_dst(pk_b, j, d128)
        pltpu.sync_copy(ones_v, deg_sp.at[d128.at[0]], add=True)
        return 0
    lax.fori_loop(0, CHUNKS, _step, 0)

    plsc.subcore_barrier()

    pltpu.sync_copy(deg_sp.at[pl.ds(row0, ROWS_PER_TILE)], degv)

    def _inv(g, _):
        dv = degv[pl.ds(g * 16, 16)]
        inv16 = 1.0 / jnp.maximum(dv, 1.0)
        for r in range(16):
            v = jnp.full((16,), inv16[r], _f32)
            for c8 in range(D // 16):
                invrep[g * 16 + r, pl.ds(c8 * 16, 16)] = v
        return 0
    lax.fori_loop(0, ROWS_PER_TILE // 16, _inv, 0)

    @pl.when(cid == 0)
    def _():
        pltpu.sync_copy(invrep, invd_out.at[pl.ds(row0, ROWS_PER_TILE)])


@functools.lru_cache(maxsize=None)
def _make_deg():
    return pl.kernel(
        _deg_body,
        out_type=[jax.ShapeDtypeStruct((N_PAD, D), _f32)],
        mesh=_mesh(),
        scratch_types=[
            pltpu.VMEM((CHUNKS, CHUNK), _i32),
            pltpu.VMEM((CHUNKS, CHUNK), _i32),
            pltpu.VMEM((1, CHUNK), _i32),
            pltpu.VMEM((CHUNK,), _f32),
            pltpu.VMEM((ROWS_PER_TILE,), _f32),
            pltpu.VMEM((ROWS_PER_TILE, D), _f32),
            pltpu.VMEM_SHARED((N_PAD,), _f32),
        ],
    )


def _mean_body(p_hbm, pkI, agg_out, pk_v, rows_v, rows_w, s0, d0, s1, d1,
               acc_sp, sem, sem2):
    cid = lax.axis_index("c")
    sid = lax.axis_index("s")
    row0 = sid * ROWS_PER_TILE
    wid = sid * NC + cid

    zero16 = jnp.zeros((16,), _f32)

    def _zr(i, _):
        rows_v[i // (D // 16), pl.ds((i % (D // 16)) * 16, 16)] = zero16
        return 0
    lax.fori_loop(0, (CHUNK * D) // 16, _zr, 0)

    for k in range(ROWS_PER_TILE // CHUNK):
        pltpu.sync_copy(rows_v, acc_sp.at[pl.ds(row0 + k * CHUNK, CHUNK)])

    pltpu.sync_copy(pkI.at[wid], pk_v)

    plsc.subcore_barrier()

    _unpack_both(pk_v, 0, s0, d0)
    _unpack_both(pk_v, 1, s1, d1)
    pltpu.async_copy(p_hbm.at[s0.at[0]], rows_v, sem)
    pltpu.async_copy(p_hbm.at[s1.at[0]], rows_w, sem2)

    def _wrap(j):
        return jnp.where(j >= CHUNKS - 2, j - (CHUNKS - 2), j + 2)

    def _step(g, _):
        for buf, sm, sb, db, off in ((rows_v, sem, s0, d0, 0),
                                     (rows_w, sem2, s1, d1, 1)):
            j = 2 * g + off
            pltpu.make_async_copy(p_hbm.at[sb.at[0]], buf, sm).wait()
            pltpu.sync_copy(buf, acc_sp.at[db.at[0]], add=True)
            _unpack_both(pk_v, _wrap(j), sb, db)
            pltpu.async_copy(p_hbm.at[sb.at[0]], buf, sm)
        return 0
    lax.fori_loop(0, CHUNKS // 2, _step, 0)
    pltpu.make_async_copy(p_hbm.at[s0.at[0]], rows_v, sem).wait()
    pltpu.make_async_copy(p_hbm.at[s1.at[0]], rows_w, sem2).wait()

    plsc.subcore_barrier()

    pltpu.sync_copy(acc_sp.at[pl.ds(row0, ROWS_PER_TILE)],
                    agg_out.at[cid, pl.ds(row0, ROWS_PER_TILE)])


@functools.lru_cache(maxsize=None)
def _make_mean():
    return pl.kernel(
        _mean_body,
        out_type=[jax.ShapeDtypeStruct((NC, N_PAD, D), _f32)],
        mesh=_mesh(),
        scratch_types=[
            pltpu.VMEM((CHUNKS, CHUNK), _i32),
            pltpu.VMEM((CHUNK, D), _f32),
            pltpu.VMEM((CHUNK, D), _f32),
            pltpu.VMEM((1, CHUNK), _i32),
            pltpu.VMEM((1, CHUNK), _i32),
            pltpu.VMEM((1, CHUNK), _i32),
            pltpu.VMEM((1, CHUNK), _i32),
            pltpu.VMEM_SHARED((N_PAD, D), _f32),
            pltpu.SemaphoreType.DMA,
            pltpu.SemaphoreType.DMA,
        ],
    )


_BLK = 1000
_GRID = N // _BLK


def _mm1_body(x_ref, wst_ref, wnt_ref, b_ref, s_ref, p_ref):
    xb = x_ref[...]
    s_ref[...] = jnp.dot(xb, wst_ref[...], preferred_element_type=_f32) + b_ref[...]
    p_ref[...] = jnp.dot(xb, wnt_ref[...], preferred_element_type=_f32)


_mm1 = pl.pallas_call(
    _mm1_body,
    grid=(_GRID,),
    in_specs=[
        pl.BlockSpec((_BLK, D), lambda i: (i, 0)),
        pl.BlockSpec((D, D), lambda i: (0, 0)),
        pl.BlockSpec((D, D), lambda i: (0, 0)),
        pl.BlockSpec((1, D), lambda i: (0, 0)),
    ],
    out_specs=[pl.BlockSpec((_BLK, D), lambda i: (i, 0)),
               pl.BlockSpec((_BLK, D), lambda i: (i, 0))],
    out_shape=[jax.ShapeDtypeStruct((N, D), _f32),
               jax.ShapeDtypeStruct((N, D), _f32)],
)


def _mid_body(s1_ref, ma_ref, mb_ref, iv_ref, wst_ref, wnt_ref, b_ref,
              s_ref, p_ref):
    h = s1_ref[...] + (ma_ref[0] + mb_ref[0]) * iv_ref[...]
    h = jnp.maximum(h, 0.0)
    s_ref[...] = jnp.dot(h, wst_ref[...], preferred_element_type=_f32) + b_ref[...]
    p_ref[...] = jnp.dot(h, wnt_ref[...], preferred_element_type=_f32)


_mid = pl.pallas_call(
    _mid_body,
    grid=(_GRID,),
    in_specs=[
        pl.BlockSpec((_BLK, D), lambda i: (i, 0)),
        pl.BlockSpec((1, _BLK, D), lambda i: (0, i, 0)),
        pl.BlockSpec((1, _BLK, D), lambda i: (1, i, 0)),
        pl.BlockSpec((_BLK, D), lambda i: (i, 0)),
        pl.BlockSpec((D, D), lambda i: (0, 0)),
        pl.BlockSpec((D, D), lambda i: (0, 0)),
        pl.BlockSpec((1, D), lambda i: (0, 0)),
    ],
    out_specs=[pl.BlockSpec((_BLK, D), lambda i: (i, 0)),
               pl.BlockSpec((_BLK, D), lambda i: (i, 0))],
    out_shape=[jax.ShapeDtypeStruct((N, D), _f32),
               jax.ShapeDtypeStruct((N, D), _f32)],
)


def _fin_body(s2_ref, ma_ref, mb_ref, iv_ref, o_ref):
    o_ref[...] = s2_ref[...] + (ma_ref[0] + mb_ref[0]) * iv_ref[...]


_fin = pl.pallas_call(
    _fin_body,
    grid=(_GRID,),
    in_specs=[
        pl.BlockSpec((_BLK, D), lambda i: (i, 0)),
        pl.BlockSpec((1, _BLK, D), lambda i: (0, i, 0)),
        pl.BlockSpec((1, _BLK, D), lambda i: (1, i, 0)),
        pl.BlockSpec((_BLK, D), lambda i: (i, 0)),
    ],
    out_specs=pl.BlockSpec((_BLK, D), lambda i: (i, 0)),
    out_shape=jax.ShapeDtypeStruct((N, D), _f32),
)


def kernel(x, edge_index, W1_self, W1_neigh, b1, W2_self, W2_neigh, b2):
    ei = edge_index.astype(_i32)
    src, dst = ei[0], ei[1]
    pad = E_PAD - E
    apad = jnp.arange(pad, dtype=_i32)
    pad_src = (apad * 131) % N
    pad_dst = N + apad % (N_PAD - N)
    src_p = jnp.concatenate([src, pad_src])
    dst_p = jnp.concatenate([dst, pad_dst])
    pkI = (src_p | (dst_p << _SHIFT)).reshape(NW, CHUNKS, CHUNK)

    (invd,) = _make_deg()(pkI)
    s1, p1 = _mm1(x, W1_self.T, W1_neigh.T, b1.reshape(1, D))
    (agg1,) = _make_mean()(p1, pkI)
    s2, p2 = _mid(s1, agg1, agg1, invd, W2_self.T, W2_neigh.T,
                  b2.reshape(1, D))
    (agg2,) = _make_mean()(p2, pkI)
    return _fin(s2, agg2, agg2, invd)


# speedup vs baseline: 13.4857x; 1.0902x over previous
import functools

import jax
import jax.numpy as jnp
from jax import lax
from jax.experimental import pallas as pl
from jax.experimental.pallas import tpu as pltpu, tpu_sc as plsc

N = 10000
E = 320000
D = 128

NC = 2
NS = 16
NW = NC * NS
ROWS_PER_TILE = 640
N_PAD = NS * ROWS_PER_TILE
CHUNK = 80
CHUNKS = 126
E_TILE = CHUNKS * CHUNK
E_PAD = NW * E_TILE
_SHIFT = 14
_MASK = (1 << _SHIFT) - 1

_f32 = jnp.float32
_i32 = jnp.int32


def _mesh():
    return plsc.VectorSubcoreMesh(core_axis_name="c", subcore_axis_name="s",
                                  num_cores=NC, num_subcores=NS)


def _unpack_both(pk, j, sbuf, dbuf):
    for c in range(CHUNK // 16):
        sl = pl.ds(c * 16, 16)
        v = pk[j, sl]
        sbuf[0, sl] = v & _MASK
        dbuf[0, sl] = lax.shift_right_logical(v, _SHIFT)


def _mean_body(with_deg, p_hbm, pkI, agg_out, degp_out, pk_v, r0, r1, r2,
               s0, d0, s1, d1, s2, d2, ones_v, zd, acc_sp, deg_sp,
               g0, g1, g2, t0, t1, t2, dsem):
    rows = (r0, r1, r2)
    sv = (s0, s1, s2)
    dv = (d0, d1, d2)
    gsem = (g0, g1, g2)
    ssem = (t0, t1, t2)

    cid = lax.axis_index("c")
    sid = lax.axis_index("s")
    row0 = sid * ROWS_PER_TILE
    wid = sid * NC + cid

    zero16 = jnp.zeros((16,), _f32)

    def _zr(i, _):
        r0[i // (D // 16), pl.ds((i % (D // 16)) * 16, 16)] = zero16
        return 0
    lax.fori_loop(0, (CHUNK * D) // 16, _zr, 0)

    for k in range(ROWS_PER_TILE // CHUNK):
        pltpu.sync_copy(r0, acc_sp.at[pl.ds(row0 + k * CHUNK, CHUNK)])

    if with_deg:
        one16 = jnp.ones((16,), _f32)

        def _zo(i, _):
            zd[pl.ds(i * 16, 16)] = zero16
            ones_v[0, pl.ds(i * 16, 16)] = one16
            return 0
        lax.fori_loop(0, CHUNK // 16, _zo, 0)
        for k in range(ROWS_PER_TILE // CHUNK):
            pltpu.sync_copy(zd, deg_sp.at[pl.ds(row0 + k * CHUNK, CHUNK)])

    pltpu.sync_copy(pkI.at[wid], pk_v)
    for u in range(3):
        _unpack_both(pk_v, u, sv[u], dv[u])

    plsc.subcore_barrier()

    pltpu.async_copy(p_hbm.at[s0.at[0]], r0, g0)
    pltpu.async_copy(p_hbm.at[s1.at[0]], r1, g1)

    def _wrap(j):
        return jnp.where(j >= CHUNKS - 2, j - (CHUNKS - 2), j + 2)

    def _step(g, _):
        for u in range(3):
            j = 3 * g + u
            b = u
            n = (u + 2) % 3
            pltpu.make_async_copy(p_hbm.at[sv[b].at[0]], rows[b],
                                  gsem[b]).wait()
            pltpu.async_copy(rows[b], acc_sp.at[dv[b].at[0]], ssem[b],
                             add=True)
            if with_deg:
                pltpu.async_copy(ones_v.at[0], deg_sp.at[dv[b].at[0]], dsem,
                                 add=True)

            @pl.when(j > 0)
            def _():
                pltpu.make_async_copy(
                    rows[n], acc_sp.at[dv[n].at[0]], ssem[n]).wait()
                if with_deg:
                    pltpu.make_async_copy(
                        ones_v.at[0], deg_sp.at[dv[n].at[0]], dsem).wait()
            _unpack_both(pk_v, _wrap(j), sv[n], dv[n])
            pltpu.async_copy(p_hbm.at[sv[n].at[0]], rows[n], gsem[n])
        return 0
    lax.fori_loop(0, CHUNKS // 3, _step, 0)

    lastb = (CHUNKS - 1) % 3
    pltpu.make_async_copy(rows[lastb], acc_sp.at[dv[lastb].at[0]],
                          ssem[lastb]).wait()
    if with_deg:
        pltpu.make_async_copy(ones_v.at[0], deg_sp.at[dv[lastb].at[0]],
                              dsem).wait()
    pltpu.make_async_copy(p_hbm.at[s0.at[0]], r0, g0).wait()
    pltpu.make_async_copy(p_hbm.at[s1.at[0]], r1, g1).wait()

    plsc.subcore_barrier()

    pltpu.sync_copy(acc_sp.at[pl.ds(row0, ROWS_PER_TILE)],
                    agg_out.at[cid, pl.ds(row0, ROWS_PER_TILE)])
    if with_deg:
        pltpu.sync_copy(deg_sp.at[pl.ds(row0, ROWS_PER_TILE)],
                        degp_out.at[cid, pl.ds(row0, ROWS_PER_TILE)])


@functools.lru_cache(maxsize=None)
def _make_mean(with_deg):
    out_type = [jax.ShapeDtypeStruct((NC, N_PAD, D), _f32)]
    if with_deg:
        out_type.append(jax.ShapeDtypeStruct((NC, N_PAD), _f32))

    if with_deg:
        def body(p_hbm, pkI, agg_out, degp_out, *s):
            _mean_body(True, p_hbm, pkI, agg_out, degp_out, *s)
    else:
        def body(p_hbm, pkI, agg_out, *s):
            _mean_body(False, p_hbm, pkI, agg_out, None, *s)

    return pl.kernel(
        body,
        out_type=out_type,
        mesh=_mesh(),
        scratch_types=[
            pltpu.VMEM((CHUNKS, CHUNK), _i32),
            pltpu.VMEM((CHUNK, D), _f32),
            pltpu.VMEM((CHUNK, D), _f32),
            pltpu.VMEM((CHUNK, D), _f32),
            pltpu.VMEM((1, CHUNK), _i32),
            pltpu.VMEM((1, CHUNK), _i32),
            pltpu.VMEM((1, CHUNK), _i32),
            pltpu.VMEM((1, CHUNK), _i32),
            pltpu.VMEM((1, CHUNK), _i32),
            pltpu.VMEM((1, CHUNK), _i32),
            pltpu.VMEM((1, CHUNK), _f32),
            pltpu.VMEM((CHUNK,), _f32),
            pltpu.VMEM_SHARED((N_PAD, D), _f32),
            pltpu.VMEM_SHARED((N_PAD,), _f32),
            pltpu.SemaphoreType.DMA,
            pltpu.SemaphoreType.DMA,
            pltpu.SemaphoreType.DMA,
            pltpu.SemaphoreType.DMA,
            pltpu.SemaphoreType.DMA,
            pltpu.SemaphoreType.DMA,
            pltpu.SemaphoreType.DMA,
        ],
    )


_BLK = 1000
_GRID = N // _BLK


def _mm1_body(x_ref, wst_ref, wnt_ref, b_ref, s_ref, p_ref):
    xb = x_ref[...]
    s_ref[...] = jnp.dot(xb, wst_ref[...], preferred_element_type=_f32) + b_ref[...]
    p_ref[...] = jnp.dot(xb, wnt_ref[...], preferred_element_type=_f32)


_mm1 = pl.pallas_call(
    _mm1_body,
    grid=(_GRID,),
    in_specs=[
        pl.BlockSpec((_BLK, D), lambda i: (i, 0)),
        pl.BlockSpec((D, D), lambda i: (0, 0)),
        pl.BlockSpec((D, D), lambda i: (0, 0)),
        pl.BlockSpec((1, D), lambda i: (0, 0)),
    ],
    out_specs=[pl.BlockSpec((_BLK, D), lambda i: (i, 0)),
               pl.BlockSpec((_BLK, D), lambda i: (i, 0))],
    out_shape=[jax.ShapeDtypeStruct((N, D), _f32),
               jax.ShapeDtypeStruct((N, D), _f32)],
)


def _inv_deg(da_ref, db_ref):
    return 1.0 / jnp.maximum(da_ref[0] + db_ref[0], 1.0)


def _mid_body(s1_ref, ma_ref, mb_ref, da_ref, db_ref, wst_ref, wnt_ref,
              b_ref, s_ref, p_ref):
    h = s1_ref[...] + (ma_ref[0] + mb_ref[0]) * _inv_deg(da_ref, db_ref)
    h = jnp.maximum(h, 0.0)
    s_ref[...] = jnp.dot(h, wst_ref[...], preferred_element_type=_f32) + b_ref[...]
    p_ref[...] = jnp.dot(h, wnt_ref[...], preferred_element_type=_f32)


_mid = pl.pallas_call(
    _mid_body,
    grid=(_GRID,),
    in_specs=[
        pl.BlockSpec((_BLK, D), lambda i: (i, 0)),
        pl.BlockSpec((1, _BLK, D), lambda i: (0, i, 0)),
        pl.BlockSpec((1, _BLK, D), lambda i: (1, i, 0)),
        pl.BlockSpec((1, _BLK, 1), lambda i: (0, i, 0)),
        pl.BlockSpec((1, _BLK, 1), lambda i: (1, i, 0)),
        pl.BlockSpec((D, D), lambda i: (0, 0)),
        pl.BlockSpec((D, D), lambda i: (0, 0)),
        pl.BlockSpec((1, D), lambda i: (0, 0)),
    ],
    out_specs=[pl.BlockSpec((_BLK, D), lambda i: (i, 0)),
               pl.BlockSpec((_BLK, D), lambda i: (i, 0))],
    out_shape=[jax.ShapeDtypeStruct((N, D), _f32),
               jax.ShapeDtypeStruct((N, D), _f32)],
)


def _fin_body(s2_ref, ma_ref, mb_ref, da_ref, db_ref, o_ref):
    o_ref[...] = s2_ref[...] + (ma_ref[0] + mb_ref[0]) * _inv_deg(da_ref, db_ref)


_fin = pl.pallas_call(
    _fin_body,
    grid=(_GRID,),
    in_specs=[
        pl.BlockSpec((_BLK, D), lambda i: (i, 0)),
        pl.BlockSpec((1, _BLK, D), lambda i: (0, i, 0)),
        pl.BlockSpec((1, _BLK, D), lambda i: (1, i, 0)),
        pl.BlockSpec((1, _BLK, 1), lambda i: (0, i, 0)),
        pl.BlockSpec((1, _BLK, 1), lambda i: (1, i, 0)),
    ],
    out_specs=pl.BlockSpec((_BLK, D), lambda i: (i, 0)),
    out_shape=jax.ShapeDtypeStruct((N, D), _f32),
)


def kernel(x, edge_index, W1_self, W1_neigh, b1, W2_self, W2_neigh, b2):
    ei = edge_index.astype(_i32)
    src, dst = ei[0], ei[1]
    pad = E_PAD - E
    apad = jnp.arange(pad, dtype=_i32)
    pad_src = (apad * 131) % N
    pad_dst = N + apad % (N_PAD - N)
    src_p = jnp.concatenate([src, pad_src])
    dst_p = jnp.concatenate([dst, pad_dst])
    pkI = (src_p | (dst_p << _SHIFT)).reshape(NW, CHUNKS, CHUNK)

    s1, p1 = _mm1(x, W1_self.T, W1_neigh.T, b1.reshape(1, D))
    agg1, degp = _make_mean(True)(p1, pkI)
    dg = degp.reshape(NC, N_PAD, 1)
    s2, p2 = _mid(s1, agg1, agg1, dg, dg, W2_self.T, W2_neigh.T,
                  b2.reshape(1, D))
    (agg2,) = _make_mean(False)(p2, pkI)
    return _fin(s2, agg2, agg2, dg, dg)
